# baseline (speedup 1.0000x reference)
import functools
import math

import jax
import jax.numpy as jnp
from jax import lax
from jax.experimental import pallas as pl
from jax.experimental.pallas import tpu as pltpu
from jax.experimental.pallas import tpu_sc as plsc

D = 384
L = 8
H = 12
DH = D // H
INTER = 768
E = 8
NLAB = 4
B = 2
S = 2048
T = B * S
RB = 256
BLK = 512
NBLK = 24
NPAD = NBLK * BLK
NA = 2 * T
QB = 512
KB = 1024

_NW = 32
_CH = NA // _NW


def _ln(x, g, b, eps=1e-5):
    m = jnp.mean(x, axis=-1, keepdims=True)
    v = jnp.mean((x - m) ** 2, axis=-1, keepdims=True)
    return (x - m) * lax.rsqrt(v + eps) * g + b


def _matmul_t(x, w):
    return lax.dot_general(x, w, (((1,), (1,)), ((), ())),
                           preferred_element_type=jnp.float32)


def _pre_body(x_ref, g_ref, b_ref, w_ref, bias_ref, o_ref):
    xn = _ln(x_ref[...], g_ref[...], b_ref[...])
    o_ref[...] = _matmul_t(xn, w_ref[...]) + bias_ref[...]


def _pre(h, g, b, w, bias):
    return pl.pallas_call(
        _pre_body,
        grid=(T // RB,),
        in_specs=[
            pl.BlockSpec((RB, D), lambda i: (i, 0)),
            pl.BlockSpec((1, D), lambda i: (0, 0)),
            pl.BlockSpec((1, D), lambda i: (0, 0)),
            pl.BlockSpec((3 * D, D), lambda i: (0, 0)),
            pl.BlockSpec((1, 3 * D), lambda i: (0, 0)),
        ],
        out_specs=pl.BlockSpec((RB, 3 * D), lambda i: (i, 0)),
        out_shape=jax.ShapeDtypeStruct((T, 3 * D), jnp.float32),
    )(h, g.reshape(1, D), b.reshape(1, D), w, bias.reshape(1, 3 * D))


HG = 128 // DH
NG = H // HG


def _attn_body(q_ref, k_ref, v_ref, o_ref):
    scale = 1.0 / math.sqrt(DH)
    qg = q_ref[0]
    outs = []
    for hh in range(HG):
        q = qg[:, hh * DH:(hh + 1) * DH]
        m = jnp.full((QB, 1), -1e30, jnp.float32)
        l = jnp.zeros((QB, 1), jnp.float32)
        acc = jnp.zeros((QB, DH), jnp.float32)
        for kb in range(S // KB):
            k = k_ref[0, kb * KB:(kb + 1) * KB, hh * DH:(hh + 1) * DH]
            v = v_ref[0, kb * KB:(kb + 1) * KB, hh * DH:(hh + 1) * DH]
            s = _matmul_t(q, k) * scale
            m_new = jnp.maximum(m, jnp.max(s, axis=-1, keepdims=True))
            p = jnp.exp(s - m_new)
            corr = jnp.exp(m - m_new)
            l = l * corr + jnp.sum(p, axis=-1, keepdims=True)
            acc = acc * corr + lax.dot_general(
                p, v, (((1,), (0,)), ((), ())),
                preferred_element_type=jnp.float32)
            m = m_new
        outs.append(acc / l)
    o_ref[0] = jnp.concatenate(outs, axis=-1)


def _attn(qkv):
    return pl.pallas_call(
        _attn_body,
        grid=(B, NG, S // QB),
        in_specs=[
            pl.BlockSpec((1, QB, 128), lambda b, g, i: (b, i, g)),
            pl.BlockSpec((1, S, 128), lambda b, g, i: (b, 0, NG + g)),
            pl.BlockSpec((1, S, 128), lambda b, g, i: (b, 0, 2 * NG + g)),
        ],
        out_specs=pl.BlockSpec((1, QB, 128), lambda b, g, i: (b, i, g)),
        out_shape=jax.ShapeDtypeStruct((B, S, D), jnp.float32),
    )(qkv, qkv, qkv)


def _post_body(a_ref, h_ref, wo_ref, bo_ref, g_ref, b_ref, gate_ref,
               hn_ref, lg_ref):
    h2 = h_ref[...] + _matmul_t(a_ref[...], wo_ref[...]) + bo_ref[...]
    hn = _ln(h2, g_ref[...], b_ref[...])
    hn_ref[...] = hn
    lg_ref[...] = lax.dot_general(gate_ref[...], hn, (((1,), (1,)), ((), ())),
                                  preferred_element_type=jnp.float32)


def _post(a, h, wo, bo, g, b, gate):
    return pl.pallas_call(
        _post_body,
        grid=(T // RB,),
        in_specs=[
            pl.BlockSpec((RB, D), lambda i: (i, 0)),
            pl.BlockSpec((RB, D), lambda i: (i, 0)),
            pl.BlockSpec((D, D), lambda i: (0, 0)),
            pl.BlockSpec((1, D), lambda i: (0, 0)),
            pl.BlockSpec((1, D), lambda i: (0, 0)),
            pl.BlockSpec((1, D), lambda i: (0, 0)),
            pl.BlockSpec((E, D), lambda i: (0, 0)),
        ],
        out_specs=[
            pl.BlockSpec((RB, D), lambda i: (i, 0)),
            pl.BlockSpec((E, RB), lambda i: (0, i)),
        ],
        out_shape=[
            jax.ShapeDtypeStruct((T, D), jnp.float32),
            jax.ShapeDtypeStruct((E, T), jnp.float32),
        ],
    )(a, h, wo, bo.reshape(1, D), g.reshape(1, D), b.reshape(1, D), gate)


def _cumsum_lanes(c, n):
    sh = 1
    while sh < n:
        z = jnp.zeros(c.shape[:-1] + (sh,), c.dtype)
        c = c + jnp.concatenate([z, c[..., :-sh]], axis=-1)
        sh *= 2
    return c


def _cumsum_rows(c, n):
    sh = 1
    while sh < n:
        z = jnp.zeros((sh,) + c.shape[1:], c.dtype)
        c = c + jnp.concatenate([z, c[:-sh]], axis=0)
        sh *= 2
    return c


def _route_body(lg_ref, dest_ref, w_ref, be_ref, nb_ref, aux_ref):
    lg = lg_ref[...]
    m = jnp.max(lg, axis=0, keepdims=True)
    ex = jnp.exp(lg - m)
    rw = ex / jnp.sum(ex, axis=0, keepdims=True)

    m1 = jnp.max(rw, axis=0, keepdims=True)
    eq1 = (rw == m1)
    c1 = _cumsum_rows(eq1.astype(jnp.int32), E)
    oh1 = eq1 & (c1 == 1)
    rw2 = jnp.where(oh1, -1.0, rw)
    m2 = jnp.max(rw2, axis=0, keepdims=True)
    eq2 = (rw2 == m2)
    c2 = _cumsum_rows(eq2.astype(jnp.int32), E)
    oh2 = eq2 & (c2 == 1)

    ssum = m1 + m2
    w0 = (m1 / ssum)[0]
    w1 = (m2 / ssum)[0]
    w_ref[:, 0:1] = w0.reshape(T, 1)
    w_ref[:, 1:2] = w1.reshape(T, 1)

    cnt = oh1.astype(jnp.int32) + oh2.astype(jnp.int32)
    cinc = _cumsum_lanes(cnt, T)
    rank = cinc - cnt
    counts = cinc[:, T - 1:T]

    padded = ((counts + (BLK - 1)) // BLK) * BLK
    cpad = _cumsum_rows(padded, E)
    offs = cpad - padded

    val = offs + rank
    dest0 = jnp.sum(jnp.where(oh1, val, 0), axis=0)
    dest1 = jnp.sum(jnp.where(oh2, val, 0), axis=0)
    dest_ref[0, :] = dest0
    dest_ref[1, :] = dest1

    bstart = lax.broadcasted_iota(jnp.int32, (1, NBLK), 1) * BLK
    be = jnp.sum((bstart >= cpad).astype(jnp.int32), axis=0, keepdims=True)
    be_ref[...] = jnp.minimum(be, E - 1)
    nb_ref[...] = (jnp.sum(padded, keepdims=True) // BLK).reshape(1, 1)

    usage = counts.astype(jnp.float32) / jnp.float32(T)
    aux_ref[...] = jnp.mean((usage - 1.0 / E) ** 2, keepdims=True).reshape(1, 1)


def _route(lg):
    return pl.pallas_call(
        _route_body,
        out_shape=[
            jax.ShapeDtypeStruct((2, T), jnp.int32),
            jax.ShapeDtypeStruct((T, 2), jnp.float32),
            jax.ShapeDtypeStruct((1, NBLK), jnp.int32),
            jax.ShapeDtypeStruct((1, 1), jnp.int32),
            jax.ShapeDtypeStruct((1, 1), jnp.float32),
        ],
    )(lg)


def _sc_dispatch(hn, dest_flat):
    mesh = plsc.VectorSubcoreMesh(core_axis_name="c", subcore_axis_name="s")

    @functools.partial(
        pl.kernel, mesh=mesh,
        out_type=jax.ShapeDtypeStruct((NPAD, D), jnp.float32),
        scratch_types=[
            pltpu.VMEM((_CH,), jnp.int32),
            pltpu.VMEM((_CH, D), jnp.float32),
            pltpu.SemaphoreType.DMA,
        ],
    )
    def k(hn_hbm, dest_hbm, buf_hbm, idx_v, rows_v, sem):
        wid = lax.axis_index("s") * 2 + lax.axis_index("c")
        base = wid * _CH
        pltpu.sync_copy(dest_hbm.at[pl.ds(base, _CH)], idx_v)
        pltpu.sync_copy(hn_hbm.at[pl.ds(lax.rem(base, T), _CH)], rows_v)
        pltpu.async_copy(rows_v, buf_hbm.at[idx_v], sem).wait()

    return k(hn, dest_flat)


def _sc_combine(buf, dest_flat):
    mesh = plsc.VectorSubcoreMesh(core_axis_name="c", subcore_axis_name="s")

    @functools.partial(
        pl.kernel, mesh=mesh,
        out_type=jax.ShapeDtypeStruct((NA, D), jnp.float32),
        scratch_types=[
            pltpu.VMEM((_CH,), jnp.int32),
            pltpu.VMEM((_CH, D), jnp.float32),
            pltpu.SemaphoreType.DMA,
        ],
    )
    def k(buf_hbm, dest_hbm, out_hbm, idx_v, rows_v, sem):
        wid = lax.axis_index("s") * 2 + lax.axis_index("c")
        base = wid * _CH
        pltpu.sync_copy(dest_hbm.at[pl.ds(base, _CH)], idx_v)
        pltpu.async_copy(buf_hbm.at[idx_v], rows_v, sem).wait()
        pltpu.sync_copy(rows_v, out_hbm.at[pl.ds(base, _CH)])

    return k(buf, dest_flat)


def _ffn_body(be_ref, nb_ref, x_ref, w1_ref, b1_ref, w2_ref, b2_ref, o_ref):
    @pl.when(pl.program_id(0) < nb_ref[0])
    def _():
        x = x_ref[...]
        mid = _matmul_t(x, w1_ref[0]) + b1_ref[0]
        mid = 0.5 * mid * (1.0 + lax.erf(mid * (1.0 / math.sqrt(2.0))))
        o_ref[...] = _matmul_t(mid, w2_ref[0]) + b2_ref[0]


def _ffn(buf, w1, b1, w2, b2, be, nb):
    grid_spec = pltpu.PrefetchScalarGridSpec(
        num_scalar_prefetch=2,
        grid=(NBLK,),
        in_specs=[
            pl.BlockSpec((BLK, D), lambda i, be, nb: (i, 0)),
            pl.BlockSpec((1, INTER, D), lambda i, be, nb: (be[i], 0, 0)),
            pl.BlockSpec((1, 1, INTER), lambda i, be, nb: (be[i], 0, 0)),
            pl.BlockSpec((1, D, INTER), lambda i, be, nb: (be[i], 0, 0)),
            pl.BlockSpec((1, 1, D), lambda i, be, nb: (be[i], 0, 0)),
        ],
        out_specs=pl.BlockSpec((BLK, D), lambda i, be, nb: (i, 0)),
    )
    return pl.pallas_call(
        _ffn_body,
        grid_spec=grid_spec,
        out_shape=jax.ShapeDtypeStruct((NPAD, D), jnp.float32),
    )(be, nb, buf, w1, b1.reshape(E, 1, INTER), w2, b2.reshape(E, 1, D))


def _combine_body(r0_ref, r1_ref, w_ref, h2_ref, mask_ref, g_ref, b_ref, o_ref):
    w = w_ref[...]
    moe = (r0_ref[0] * w[:, 0:1] + r1_ref[0] * w[:, 1:2]) * mask_ref[...]
    o_ref[...] = _ln(h2_ref[...] + moe, g_ref[...], b_ref[...])


def _combine(comb, w, h2, mask, g, b):
    comb3 = comb.reshape(2, T, D)
    return pl.pallas_call(
        _combine_body,
        grid=(T // RB,),
        in_specs=[
            pl.BlockSpec((1, RB, D), lambda i: (0, i, 0)),
            pl.BlockSpec((1, RB, D), lambda i: (1, i, 0)),
            pl.BlockSpec((RB, 2), lambda i: (i, 0)),
            pl.BlockSpec((RB, D), lambda i: (i, 0)),
            pl.BlockSpec((RB, 1), lambda i: (i, 0)),
            pl.BlockSpec((1, D), lambda i: (0, 0)),
            pl.BlockSpec((1, D), lambda i: (0, 0)),
        ],
        out_specs=pl.BlockSpec((RB, D), lambda i: (i, 0)),
        out_shape=jax.ShapeDtypeStruct((T, D), jnp.float32),
    )(comb3, comb3, w, h2, mask, g.reshape(1, D), b.reshape(1, D))


def _combine_pre_body(r0_ref, r1_ref, w_ref, h2_ref, mask_ref, g_ref, b_ref,
                      g1_ref, b1_ref, wq_ref, bq_ref, o_ref, qkv_ref):
    w = w_ref[...]
    moe = (r0_ref[0] * w[:, 0:1] + r1_ref[0] * w[:, 1:2]) * mask_ref[...]
    h = _ln(h2_ref[...] + moe, g_ref[...], b_ref[...])
    o_ref[...] = h
    xn = _ln(h, g1_ref[...], b1_ref[...])
    qkv_ref[...] = _matmul_t(xn, wq_ref[...]) + bq_ref[...]


def _combine_pre(comb, w, h2, mask, g, b, g1, b1, wq, bq):
    comb3 = comb.reshape(2, T, D)
    return pl.pallas_call(
        _combine_pre_body,
        grid=(T // RB,),
        in_specs=[
            pl.BlockSpec((1, RB, D), lambda i: (0, i, 0)),
            pl.BlockSpec((1, RB, D), lambda i: (1, i, 0)),
            pl.BlockSpec((RB, 2), lambda i: (i, 0)),
            pl.BlockSpec((RB, D), lambda i: (i, 0)),
            pl.BlockSpec((RB, 1), lambda i: (i, 0)),
            pl.BlockSpec((1, D), lambda i: (0, 0)),
            pl.BlockSpec((1, D), lambda i: (0, 0)),
            pl.BlockSpec((1, D), lambda i: (0, 0)),
            pl.BlockSpec((1, D), lambda i: (0, 0)),
            pl.BlockSpec((3 * D, D), lambda i: (0, 0)),
            pl.BlockSpec((1, 3 * D), lambda i: (0, 0)),
        ],
        out_specs=[
            pl.BlockSpec((RB, D), lambda i: (i, 0)),
            pl.BlockSpec((RB, 3 * D), lambda i: (i, 0)),
        ],
        out_shape=[
            jax.ShapeDtypeStruct((T, D), jnp.float32),
            jax.ShapeDtypeStruct((T, 3 * D), jnp.float32),
        ],
    )(comb3, comb3, w, h2, mask, g.reshape(1, D), b.reshape(1, D),
      g1.reshape(1, D), b1.reshape(1, D), wq, bq.reshape(1, 3 * D))


def _head_body(x_ref, fg_ref, fb_ref, pw_ref, pb_ref, cw_ref, cb_ref,
               c1w_ref, c1b_ref, c2w_ref, c2b_ref, lg_ref, cf_ref):
    x = _ln(x_ref[...], fg_ref[...], fb_ref[...])
    pooled = jnp.tanh(_matmul_t(x, pw_ref[...]) + pb_ref[...])
    lg_ref[...] = _matmul_t(pooled, cw_ref[...]) + cb_ref[...]
    c1 = jnp.maximum(_matmul_t(pooled, c1w_ref[...]) + c1b_ref[...], 0.0)
    z = _matmul_t(c1, c2w_ref[...]) + c2b_ref[...]
    cf_ref[...] = 1.0 / (1.0 + jnp.exp(-z))


def _head(hcls, p):
    return pl.pallas_call(
        _head_body,
        out_shape=[
            jax.ShapeDtypeStruct((B, NLAB), jnp.float32),
            jax.ShapeDtypeStruct((B, 8), jnp.float32),
        ],
    )(hcls, p['fng'].reshape(1, D), p['fnb'].reshape(1, D),
      p['poolW'], p['poolb'].reshape(1, D),
      p['clsW'], p['clsb'].reshape(1, NLAB),
      p['cW1'], p['cb1'].reshape(1, D // 2),
      jnp.pad(p['cW2'], ((0, 7), (0, 0))),
      jnp.pad(p['cb2'].reshape(1, 1), ((0, 0), (0, 7))))


def kernel(embeddings, attention_mask, params):
    p = params
    h = embeddings.astype(jnp.float32).reshape(T, D)
    mask = attention_mask.astype(jnp.float32).reshape(T, 1)
    aux_total = jnp.float32(0.0)
    qkv = _pre(h, p['n1g'][0], p['n1b'][0], p['Wqkv'][0], p['bqkv'][0])
    for l in range(L):
        a = _attn(qkv.reshape(B, S, 3 * D)).reshape(T, D)
        hn, lg = _post(a, h, p['Wo'][l], p['bo'][l],
                       p['n2g'][l], p['n2b'][l], p['gate'][l])
        dest, w, be, nb, auxl = _route(lg)
        dest_flat = dest.reshape(NA)
        buf = _sc_dispatch(hn, dest_flat)
        fout = _ffn(buf, p['W1'][l], p['b1'][l], p['W2'][l], p['b2'][l],
                    be.reshape(NBLK), nb.reshape(1))
        comb = _sc_combine(fout, dest_flat)
        if l < L - 1:
            h, qkv = _combine_pre(comb, w, hn, mask, p['mng'][l], p['mnb'][l],
                                  p['n1g'][l + 1], p['n1b'][l + 1],
                                  p['Wqkv'][l + 1], p['bqkv'][l + 1])
        else:
            h = _combine(comb, w, hn, mask, p['mng'][l], p['mnb'][l])
        aux_total = aux_total + auxl[0, 0]
    hcls = h.reshape(B, S, D)[:, 0]
    logits, conf = _head(hcls, p)
    return (logits, aux_total, conf[:, :1])


# speedup vs baseline: 1.0473x; 1.0473x over previous
import functools
import math

import jax
import jax.numpy as jnp
from jax import lax
from jax.experimental import pallas as pl
from jax.experimental.pallas import tpu as pltpu
from jax.experimental.pallas import tpu_sc as plsc

D = 384
L = 8
H = 12
DH = D // H
INTER = 768
E = 8
NLAB = 4
B = 2
S = 2048
T = B * S
RB = 256
BLK = 512
NBLK = 24
NPAD = NBLK * BLK
NA = 2 * T
QB = 512
KB = 2048

_NW = 32
_CH = NA // _NW


def _ln(x, g, b, eps=1e-5):
    m = jnp.mean(x, axis=-1, keepdims=True)
    v = jnp.mean((x - m) ** 2, axis=-1, keepdims=True)
    return (x - m) * lax.rsqrt(v + eps) * g + b


def _matmul_t(x, w):
    return lax.dot_general(x, w, (((1,), (1,)), ((), ())),
                           preferred_element_type=jnp.float32)


def _pre_body(x_ref, g_ref, b_ref, w_ref, bias_ref, o_ref):
    xn = _ln(x_ref[...], g_ref[...], b_ref[...])
    o_ref[...] = _matmul_t(xn, w_ref[...]) + bias_ref[...]


def _pre(h, g, b, w, bias):
    return pl.pallas_call(
        _pre_body,
        grid=(T // RB,),
        in_specs=[
            pl.BlockSpec((RB, D), lambda i: (i, 0)),
            pl.BlockSpec((1, D), lambda i: (0, 0)),
            pl.BlockSpec((1, D), lambda i: (0, 0)),
            pl.BlockSpec((3 * D, D), lambda i: (0, 0)),
            pl.BlockSpec((1, 3 * D), lambda i: (0, 0)),
        ],
        out_specs=pl.BlockSpec((RB, 3 * D), lambda i: (i, 0)),
        out_shape=jax.ShapeDtypeStruct((T, 3 * D), jnp.float32),
    )(h, g.reshape(1, D), b.reshape(1, D), w, bias.reshape(1, 3 * D))


HG = 128 // DH
NG = H // HG


def _attn_body(q_ref, k_ref, v_ref, o_ref):
    scale = 1.0 / math.sqrt(DH)
    qg = q_ref[0]
    outs = []
    for hh in range(HG):
        q = qg[:, hh * DH:(hh + 1) * DH]
        m = jnp.full((QB, 1), -1e30, jnp.float32)
        l = jnp.zeros((QB, 1), jnp.float32)
        acc = jnp.zeros((QB, DH), jnp.float32)
        for kb in range(S // KB):
            k = k_ref[0, kb * KB:(kb + 1) * KB, hh * DH:(hh + 1) * DH]
            v = v_ref[0, kb * KB:(kb + 1) * KB, hh * DH:(hh + 1) * DH]
            s = _matmul_t(q, k) * scale
            m_new = jnp.maximum(m, jnp.max(s, axis=-1, keepdims=True))
            p = jnp.exp(s - m_new)
            corr = jnp.exp(m - m_new)
            l = l * corr + jnp.sum(p, axis=-1, keepdims=True)
            acc = acc * corr + lax.dot_general(
                p, v, (((1,), (0,)), ((), ())),
                preferred_element_type=jnp.float32)
            m = m_new
        outs.append(acc / l)
    o_ref[0] = jnp.concatenate(outs, axis=-1)


def _attn(qkv):
    return pl.pallas_call(
        _attn_body,
        grid=(B, NG, S // QB),
        in_specs=[
            pl.BlockSpec((1, QB, 128), lambda b, g, i: (b, i, g)),
            pl.BlockSpec((1, S, 128), lambda b, g, i: (b, 0, NG + g)),
            pl.BlockSpec((1, S, 128), lambda b, g, i: (b, 0, 2 * NG + g)),
        ],
        out_specs=pl.BlockSpec((1, QB, 128), lambda b, g, i: (b, i, g)),
        out_shape=jax.ShapeDtypeStruct((B, S, D), jnp.float32),
    )(qkv, qkv, qkv)


def _post_body(a_ref, h_ref, wo_ref, bo_ref, g_ref, b_ref, gate_ref,
               hn_ref, lg_ref):
    h2 = h_ref[...] + _matmul_t(a_ref[...], wo_ref[...]) + bo_ref[...]
    hn = _ln(h2, g_ref[...], b_ref[...])
    hn_ref[...] = hn
    lg_ref[...] = lax.dot_general(gate_ref[...], hn, (((1,), (1,)), ((), ())),
                                  preferred_element_type=jnp.float32)


def _post(a, h, wo, bo, g, b, gate):
    return pl.pallas_call(
        _post_body,
        grid=(T // RB,),
        in_specs=[
            pl.BlockSpec((RB, D), lambda i: (i, 0)),
            pl.BlockSpec((RB, D), lambda i: (i, 0)),
            pl.BlockSpec((D, D), lambda i: (0, 0)),
            pl.BlockSpec((1, D), lambda i: (0, 0)),
            pl.BlockSpec((1, D), lambda i: (0, 0)),
            pl.BlockSpec((1, D), lambda i: (0, 0)),
            pl.BlockSpec((E, D), lambda i: (0, 0)),
        ],
        out_specs=[
            pl.BlockSpec((RB, D), lambda i: (i, 0)),
            pl.BlockSpec((E, RB), lambda i: (0, i)),
        ],
        out_shape=[
            jax.ShapeDtypeStruct((T, D), jnp.float32),
            jax.ShapeDtypeStruct((E, T), jnp.float32),
        ],
    )(a, h, wo, bo.reshape(1, D), g.reshape(1, D), b.reshape(1, D), gate)


def _cumsum_lanes(c, n):
    sh = 1
    while sh < n:
        z = jnp.zeros(c.shape[:-1] + (sh,), c.dtype)
        c = c + jnp.concatenate([z, c[..., :-sh]], axis=-1)
        sh *= 2
    return c


def _cumsum_rows(c, n):
    sh = 1
    while sh < n:
        z = jnp.zeros((sh,) + c.shape[1:], c.dtype)
        c = c + jnp.concatenate([z, c[:-sh]], axis=0)
        sh *= 2
    return c


def _route_body(lg_ref, dest_ref, w_ref, be_ref, nb_ref, aux_ref):
    lg = lg_ref[...]
    m = jnp.max(lg, axis=0, keepdims=True)
    ex = jnp.exp(lg - m)
    rw = ex / jnp.sum(ex, axis=0, keepdims=True)

    m1 = jnp.max(rw, axis=0, keepdims=True)
    eq1 = (rw == m1)
    c1 = _cumsum_rows(eq1.astype(jnp.int32), E)
    oh1 = eq1 & (c1 == 1)
    rw2 = jnp.where(oh1, -1.0, rw)
    m2 = jnp.max(rw2, axis=0, keepdims=True)
    eq2 = (rw2 == m2)
    c2 = _cumsum_rows(eq2.astype(jnp.int32), E)
    oh2 = eq2 & (c2 == 1)

    ssum = m1 + m2
    w0 = (m1 / ssum)[0]
    w1 = (m2 / ssum)[0]
    w_ref[:, 0:1] = w0.reshape(T, 1)
    w_ref[:, 1:2] = w1.reshape(T, 1)

    cnt = oh1.astype(jnp.int32) + oh2.astype(jnp.int32)
    cinc = _cumsum_lanes(cnt, T)
    rank = cinc - cnt
    counts = cinc[:, T - 1:T]

    padded = ((counts + (BLK - 1)) // BLK) * BLK
    cpad = _cumsum_rows(padded, E)
    offs = cpad - padded

    val = offs + rank
    dest0 = jnp.sum(jnp.where(oh1, val, 0), axis=0)
    dest1 = jnp.sum(jnp.where(oh2, val, 0), axis=0)
    dest_ref[0, :] = dest0
    dest_ref[1, :] = dest1

    bstart = lax.broadcasted_iota(jnp.int32, (1, NBLK), 1) * BLK
    be = jnp.sum((bstart >= cpad).astype(jnp.int32), axis=0, keepdims=True)
    be_ref[...] = jnp.minimum(be, E - 1)
    nb_ref[...] = (jnp.sum(padded, keepdims=True) // BLK).reshape(1, 1)

    usage = counts.astype(jnp.float32) / jnp.float32(T)
    aux_ref[...] = jnp.mean((usage - 1.0 / E) ** 2, keepdims=True).reshape(1, 1)


def _route(lg):
    return pl.pallas_call(
        _route_body,
        out_shape=[
            jax.ShapeDtypeStruct((2, T), jnp.int32),
            jax.ShapeDtypeStruct((T, 2), jnp.float32),
            jax.ShapeDtypeStruct((1, NBLK), jnp.int32),
            jax.ShapeDtypeStruct((1, 1), jnp.int32),
            jax.ShapeDtypeStruct((1, 1), jnp.float32),
        ],
    )(lg)


def _sc_dispatch(hn, dest_flat):
    mesh = plsc.VectorSubcoreMesh(core_axis_name="c", subcore_axis_name="s")

    @functools.partial(
        pl.kernel, mesh=mesh,
        out_type=jax.ShapeDtypeStruct((NPAD, D), jnp.float32),
        scratch_types=[
            pltpu.VMEM((_CH,), jnp.int32),
            pltpu.VMEM((_CH, D), jnp.float32),
            pltpu.SemaphoreType.DMA,
        ],
    )
    def k(hn_hbm, dest_hbm, buf_hbm, idx_v, rows_v, sem):
        wid = lax.axis_index("s") * 2 + lax.axis_index("c")
        base = wid * _CH
        pltpu.sync_copy(dest_hbm.at[pl.ds(base, _CH)], idx_v)
        pltpu.sync_copy(hn_hbm.at[pl.ds(lax.rem(base, T), _CH)], rows_v)
        pltpu.async_copy(rows_v, buf_hbm.at[idx_v], sem).wait()

    return k(hn, dest_flat)


def _sc_combine(buf, dest_flat):
    mesh = plsc.VectorSubcoreMesh(core_axis_name="c", subcore_axis_name="s")

    @functools.partial(
        pl.kernel, mesh=mesh,
        out_type=jax.ShapeDtypeStruct((NA, D), jnp.float32),
        scratch_types=[
            pltpu.VMEM((_CH,), jnp.int32),
            pltpu.VMEM((_CH, D), jnp.float32),
            pltpu.SemaphoreType.DMA,
        ],
    )
    def k(buf_hbm, dest_hbm, out_hbm, idx_v, rows_v, sem):
        wid = lax.axis_index("s") * 2 + lax.axis_index("c")
        base = wid * _CH
        pltpu.sync_copy(dest_hbm.at[pl.ds(base, _CH)], idx_v)
        pltpu.async_copy(buf_hbm.at[idx_v], rows_v, sem).wait()
        pltpu.sync_copy(rows_v, out_hbm.at[pl.ds(base, _CH)])

    return k(buf, dest_flat)


def _ffn_body(be_ref, nb_ref, x_ref, w1_ref, b1_ref, w2_ref, b2_ref, o_ref):
    @pl.when(pl.program_id(0) < nb_ref[0])
    def _():
        x = x_ref[...]
        mid = _matmul_t(x, w1_ref[0]) + b1_ref[0]
        mid = 0.5 * mid * (1.0 + lax.erf(mid * (1.0 / math.sqrt(2.0))))
        o_ref[...] = _matmul_t(mid, w2_ref[0]) + b2_ref[0]


def _ffn(buf, w1, b1, w2, b2, be, nb):
    grid_spec = pltpu.PrefetchScalarGridSpec(
        num_scalar_prefetch=2,
        grid=(NBLK,),
        in_specs=[
            pl.BlockSpec((BLK, D), lambda i, be, nb: (i, 0)),
            pl.BlockSpec((1, INTER, D), lambda i, be, nb: (be[i], 0, 0)),
            pl.BlockSpec((1, 1, INTER), lambda i, be, nb: (be[i], 0, 0)),
            pl.BlockSpec((1, D, INTER), lambda i, be, nb: (be[i], 0, 0)),
            pl.BlockSpec((1, 1, D), lambda i, be, nb: (be[i], 0, 0)),
        ],
        out_specs=pl.BlockSpec((BLK, D), lambda i, be, nb: (i, 0)),
    )
    return pl.pallas_call(
        _ffn_body,
        grid_spec=grid_spec,
        out_shape=jax.ShapeDtypeStruct((NPAD, D), jnp.float32),
    )(be, nb, buf, w1, b1.reshape(E, 1, INTER), w2, b2.reshape(E, 1, D))


def _combine_body(r0_ref, r1_ref, w_ref, h2_ref, mask_ref, g_ref, b_ref, o_ref):
    w = w_ref[...]
    moe = (r0_ref[0] * w[:, 0:1] + r1_ref[0] * w[:, 1:2]) * mask_ref[...]
    o_ref[...] = _ln(h2_ref[...] + moe, g_ref[...], b_ref[...])


def _combine(comb, w, h2, mask, g, b):
    comb3 = comb.reshape(2, T, D)
    return pl.pallas_call(
        _combine_body,
        grid=(T // RB,),
        in_specs=[
            pl.BlockSpec((1, RB, D), lambda i: (0, i, 0)),
            pl.BlockSpec((1, RB, D), lambda i: (1, i, 0)),
            pl.BlockSpec((RB, 2), lambda i: (i, 0)),
            pl.BlockSpec((RB, D), lambda i: (i, 0)),
            pl.BlockSpec((RB, 1), lambda i: (i, 0)),
            pl.BlockSpec((1, D), lambda i: (0, 0)),
            pl.BlockSpec((1, D), lambda i: (0, 0)),
        ],
        out_specs=pl.BlockSpec((RB, D), lambda i: (i, 0)),
        out_shape=jax.ShapeDtypeStruct((T, D), jnp.float32),
    )(comb3, comb3, w, h2, mask, g.reshape(1, D), b.reshape(1, D))


def _combine_pre_body(r0_ref, r1_ref, w_ref, h2_ref, mask_ref, g_ref, b_ref,
                      g1_ref, b1_ref, wq_ref, bq_ref, o_ref, qkv_ref):
    w = w_ref[...]
    moe = (r0_ref[0] * w[:, 0:1] + r1_ref[0] * w[:, 1:2]) * mask_ref[...]
    h = _ln(h2_ref[...] + moe, g_ref[...], b_ref[...])
    o_ref[...] = h
    xn = _ln(h, g1_ref[...], b1_ref[...])
    qkv_ref[...] = _matmul_t(xn, wq_ref[...]) + bq_ref[...]


def _combine_pre(comb, w, h2, mask, g, b, g1, b1, wq, bq):
    comb3 = comb.reshape(2, T, D)
    return pl.pallas_call(
        _combine_pre_body,
        grid=(T // RB,),
        in_specs=[
            pl.BlockSpec((1, RB, D), lambda i: (0, i, 0)),
            pl.BlockSpec((1, RB, D), lambda i: (1, i, 0)),
            pl.BlockSpec((RB, 2), lambda i: (i, 0)),
            pl.BlockSpec((RB, D), lambda i: (i, 0)),
            pl.BlockSpec((RB, 1), lambda i: (i, 0)),
            pl.BlockSpec((1, D), lambda i: (0, 0)),
            pl.BlockSpec((1, D), lambda i: (0, 0)),
            pl.BlockSpec((1, D), lambda i: (0, 0)),
            pl.BlockSpec((1, D), lambda i: (0, 0)),
            pl.BlockSpec((3 * D, D), lambda i: (0, 0)),
            pl.BlockSpec((1, 3 * D), lambda i: (0, 0)),
        ],
        out_specs=[
            pl.BlockSpec((RB, D), lambda i: (i, 0)),
            pl.BlockSpec((RB, 3 * D), lambda i: (i, 0)),
        ],
        out_shape=[
            jax.ShapeDtypeStruct((T, D), jnp.float32),
            jax.ShapeDtypeStruct((T, 3 * D), jnp.float32),
        ],
    )(comb3, comb3, w, h2, mask, g.reshape(1, D), b.reshape(1, D),
      g1.reshape(1, D), b1.reshape(1, D), wq, bq.reshape(1, 3 * D))


def _head_body(x_ref, fg_ref, fb_ref, pw_ref, pb_ref, cw_ref, cb_ref,
               c1w_ref, c1b_ref, c2w_ref, c2b_ref, lg_ref, cf_ref):
    x = _ln(x_ref[...], fg_ref[...], fb_ref[...])
    pooled = jnp.tanh(_matmul_t(x, pw_ref[...]) + pb_ref[...])
    lg_ref[...] = _matmul_t(pooled, cw_ref[...]) + cb_ref[...]
    c1 = jnp.maximum(_matmul_t(pooled, c1w_ref[...]) + c1b_ref[...], 0.0)
    z = _matmul_t(c1, c2w_ref[...]) + c2b_ref[...]
    cf_ref[...] = 1.0 / (1.0 + jnp.exp(-z))


def _head(hcls, p):
    return pl.pallas_call(
        _head_body,
        out_shape=[
            jax.ShapeDtypeStruct((B, NLAB), jnp.float32),
            jax.ShapeDtypeStruct((B, 8), jnp.float32),
        ],
    )(hcls, p['fng'].reshape(1, D), p['fnb'].reshape(1, D),
      p['poolW'], p['poolb'].reshape(1, D),
      p['clsW'], p['clsb'].reshape(1, NLAB),
      p['cW1'], p['cb1'].reshape(1, D // 2),
      jnp.pad(p['cW2'], ((0, 7), (0, 0))),
      jnp.pad(p['cb2'].reshape(1, 1), ((0, 0), (0, 7))))


def kernel(embeddings, attention_mask, params):
    p = params
    h = embeddings.astype(jnp.float32).reshape(T, D)
    mask = attention_mask.astype(jnp.float32).reshape(T, 1)
    aux_total = jnp.float32(0.0)
    qkv = _pre(h, p['n1g'][0], p['n1b'][0], p['Wqkv'][0], p['bqkv'][0])
    for l in range(L):
        a = _attn(qkv.reshape(B, S, 3 * D)).reshape(T, D)
        hn, lg = _post(a, h, p['Wo'][l], p['bo'][l],
                       p['n2g'][l], p['n2b'][l], p['gate'][l])
        dest, w, be, nb, auxl = _route(lg)
        dest_flat = dest.reshape(NA)
        buf = _sc_dispatch(hn, dest_flat)
        fout = _ffn(buf, p['W1'][l], p['b1'][l], p['W2'][l], p['b2'][l],
                    be.reshape(NBLK), nb.reshape(1))
        comb = _sc_combine(fout, dest_flat)
        if l < L - 1:
            h, qkv = _combine_pre(comb, w, hn, mask, p['mng'][l], p['mnb'][l],
                                  p['n1g'][l + 1], p['n1b'][l + 1],
                                  p['Wqkv'][l + 1], p['bqkv'][l + 1])
        else:
            h = _combine(comb, w, hn, mask, p['mng'][l], p['mnb'][l])
        aux_total = aux_total + auxl[0, 0]
    hcls = h.reshape(B, S, D)[:, 0]
    logits, conf = _head(hcls, p)
    return (logits, aux_total, conf[:, :1])
